# baseline (speedup 1.0000x reference)
import functools
import jax
import jax.numpy as jnp
from jax import lax
from jax.experimental import pallas as pl
from jax.experimental.pallas import tpu as pltpu
from jax.experimental.pallas import tpu_sc as plsc

N = 10000
E = 320000
D = 128
NC = 2
NS = 16
NW = NC * NS
CH = 80
EPT = E // NW
NCH = EPT // CH
ZR = 80
NZCH = N // ZR


def _fill_const(buf, rows, cols, val):
    v = jnp.full((16,), val, jnp.float32)

    def row(r, _):
        for k in range(cols // 16):
            buf[r, pl.ds(k * 16, 16)] = v
        return 0

    lax.fori_loop(0, rows, row, 0)


def _zero_spmem(s, zbuf, agg_s):
    for j in range((NZCH + NS - 1) // NS):
        k = s + j * NS

        @pl.when(k < NZCH)
        def _():
            pltpu.sync_copy(zbuf, agg_s.at[pl.ds(k * ZR, ZR)])


def _export_spmem(c, s, zbuf, agg_s, out_hbm):
    for j in range((NZCH + NS - 1) // NS):
        k = s + j * NS

        @pl.when(k < NZCH)
        def _():
            pltpu.sync_copy(agg_s.at[pl.ds(k * ZR, ZR)], zbuf)
            pltpu.sync_copy(zbuf, out_hbm.at[pl.ds(c * N + k * ZR, ZR)])


def _agg_body(x_hbm, src_hbm, dst_hbm, out_hbm, isrc, idst, rows0, rows1,
              sem0, sem1, sem_s0, sem_s1, agg_s):
    c = lax.axis_index("c")
    s = lax.axis_index("s")
    wid = c * NS + s

    zbuf = rows0.at[pl.ds(0, ZR)]
    _fill_const(rows0, ZR, D, 0.0)
    _zero_spmem(s, zbuf, agg_s)

    pltpu.sync_copy(src_hbm.at[pl.ds(wid * EPT, EPT)], isrc)
    pltpu.sync_copy(dst_hbm.at[wid], idst)
    plsc.subcore_barrier()

    cp0 = pltpu.async_copy(x_hbm.at[isrc.at[pl.ds(0, CH)]], rows0, sem0)
    cp1 = pltpu.async_copy(x_hbm.at[isrc.at[pl.ds(CH, CH)]], rows1, sem1)

    def pair(i, _):
        k = 2 * i
        cp0.wait()
        sc0 = pltpu.async_copy(rows0, agg_s.at[idst.at[k]], sem_s0, add=True)
        cp1.wait()
        sc1 = pltpu.async_copy(rows1, agg_s.at[idst.at[k + 1]], sem_s1,
                               add=True)
        sc0.wait()

        @pl.when(k + 2 < NCH)
        def _():
            pltpu.async_copy(x_hbm.at[isrc.at[pl.ds((k + 2) * CH, CH)]],
                             rows0, sem0)

        sc1.wait()

        @pl.when(k + 3 < NCH)
        def _():
            pltpu.async_copy(x_hbm.at[isrc.at[pl.ds((k + 3) * CH, CH)]],
                             rows1, sem1)

        return 0

    lax.fori_loop(0, NCH // 2, pair, 0)
    if NCH % 2 == 1:
        cp0.wait()
        pltpu.sync_copy(rows0, agg_s.at[idst.at[NCH - 1]], add=True)
    plsc.subcore_barrier()
    _export_spmem(c, s, rows0.at[pl.ds(0, ZR)], agg_s, out_hbm)


_SC_MESH = plsc.VectorSubcoreMesh(core_axis_name="c", subcore_axis_name="s")

_agg = pl.kernel(
    _agg_body,
    out_type=(jax.ShapeDtypeStruct((NC * N, D), jnp.float32),),
    mesh=_SC_MESH,
    scratch_types=[
        pltpu.VMEM((EPT,), jnp.int32),
        pltpu.VMEM((NCH, CH), jnp.int32),
        pltpu.VMEM((CH, D), jnp.float32),
        pltpu.VMEM((CH, D), jnp.float32),
        pltpu.SemaphoreType.DMA,
        pltpu.SemaphoreType.DMA,
        pltpu.SemaphoreType.DMA,
        pltpu.SemaphoreType.DMA,
        pltpu.VMEM_SHARED((N, D), jnp.float32),
    ],
)

BN = 1000


def _dense1_body(p0, p1, d0, d1, h, wl, b, wr, o, oinv):
    deg = (d0[...] + d1[...])[:, 0:1]
    inv = 1.0 / jnp.maximum(deg, 1.0)
    oinv[...] = inv
    agg = (p0[...] + p1[...]) * inv
    acc = jnp.dot(agg, wl[...], preferred_element_type=jnp.float32)
    acc += jnp.dot(h[...], wr[...], preferred_element_type=jnp.float32)
    o[...] = jnp.maximum(acc + b[...], 0.0)


def _dense_body(relu, p0, p1, dinv, h, wl, b, wr, o):
    agg = (p0[...] + p1[...]) * dinv[...]
    acc = jnp.dot(agg, wl[...], preferred_element_type=jnp.float32)
    acc += jnp.dot(h[...], wr[...], preferred_element_type=jnp.float32)
    acc += b[...]
    o[...] = jnp.maximum(acc, 0.0) if relu else acc


def _part_specs():
    return [
        pl.BlockSpec((BN, D), lambda i: (i, 0)),
        pl.BlockSpec((BN, D), lambda i: (i + N // BN, 0)),
    ]


_W_SPECS = [
    pl.BlockSpec((D, D), lambda i: (0, 0)),
    pl.BlockSpec((1, D), lambda i: (0, 0)),
    pl.BlockSpec((D, D), lambda i: (0, 0)),
]

_dense1 = pl.pallas_call(
    _dense1_body,
    grid=(N // BN,),
    in_specs=_part_specs() + _part_specs() + [
        pl.BlockSpec((BN, D), lambda i: (i, 0)),
    ] + _W_SPECS,
    out_specs=[
        pl.BlockSpec((BN, D), lambda i: (i, 0)),
        pl.BlockSpec((BN, 1), lambda i: (i, 0)),
    ],
    out_shape=[
        jax.ShapeDtypeStruct((N, D), jnp.float32),
        jax.ShapeDtypeStruct((N, 1), jnp.float32),
    ],
)


def _make_dense(relu):
    return pl.pallas_call(
        functools.partial(_dense_body, relu),
        grid=(N // BN,),
        in_specs=_part_specs() + [
            pl.BlockSpec((BN, 1), lambda i: (i, 0)),
            pl.BlockSpec((BN, D), lambda i: (i, 0)),
        ] + _W_SPECS,
        out_specs=pl.BlockSpec((BN, D), lambda i: (i, 0)),
        out_shape=jax.ShapeDtypeStruct((N, D), jnp.float32),
    )


_dense_relu = _make_dense(True)
_dense_lin = _make_dense(False)


def kernel(x, edge_index, c1_Wl, c1_b, c1_Wr, c2_Wl, c2_b, c2_Wr,
           c3_Wl, c3_b, c3_Wr, c4a_Wl, c4a_b, c4a_Wr, c4s_Wl, c4s_b,
           c4s_Wr, c4e_Wl, c4e_b, c4e_Wr):
    src = edge_index[0]
    dst = edge_index[1].reshape(NW, NCH, CH)

    degp = _agg(jnp.ones((N, D), jnp.float32), src, dst)[0]
    parts = _agg(x, src, dst)[0]
    h1, inv = _dense1(parts, parts, degp, degp, x,
                      c1_Wl, c1_b.reshape(1, D), c1_Wr)

    parts = _agg(h1, src, dst)[0]
    h2 = _dense_relu(parts, parts, inv, h1, c2_Wl, c2_b.reshape(1, D), c2_Wr)

    parts = _agg(h2, src, dst)[0]
    h3 = _dense_relu(parts, parts, inv, h2, c3_Wl, c3_b.reshape(1, D), c3_Wr)

    F = 21 + 2 + 5
    wl4 = jnp.zeros((D, D), jnp.float32)
    wl4 = wl4.at[:, :F].set(jnp.concatenate([c4a_Wl, c4s_Wl, c4e_Wl], axis=1))
    wr4 = jnp.zeros((D, D), jnp.float32)
    wr4 = wr4.at[:, :F].set(jnp.concatenate([c4a_Wr, c4s_Wr, c4e_Wr], axis=1))
    b4 = jnp.zeros((1, D), jnp.float32)
    b4 = b4.at[0, :F].set(jnp.concatenate([c4a_b, c4s_b, c4e_b]))

    parts = _agg(h3, src, dst)[0]
    out4 = _dense_lin(parts, parts, inv, h3, wl4, b4, wr4)
    return (out4[:, :21], out4[:, 21:23], out4[:, 23:28])


# speedup vs baseline: 1.2550x; 1.2550x over previous
import functools
import jax
import jax.numpy as jnp
from jax import lax
from jax.experimental import pallas as pl
from jax.experimental.pallas import tpu as pltpu
from jax.experimental.pallas import tpu_sc as plsc

N = 10000
E = 320000
D = 128
NC = 2
NS = 16
NW = NC * NS
CH = 80
EPT = E // NW
NCH = EPT // CH
ZR = 80
NZCH = N // ZR


def _fill_const(buf, rows, cols, val):
    v = jnp.full((16,), val, jnp.float32)

    def row(r, _):
        for k in range(cols // 16):
            buf[r, pl.ds(k * 16, 16)] = v
        return 0

    lax.fori_loop(0, rows, row, 0)


def _zero_spmem(s, zbuf, agg_s):
    for j in range((NZCH + NS - 1) // NS):
        k = s + j * NS

        @pl.when(k < NZCH)
        def _():
            pltpu.sync_copy(zbuf, agg_s.at[pl.ds(k * ZR, ZR)])


def _export_spmem(c, s, zbuf, agg_s, out_hbm):
    for j in range((NZCH + NS - 1) // NS):
        k = s + j * NS

        @pl.when(k < NZCH)
        def _():
            pltpu.sync_copy(agg_s.at[pl.ds(k * ZR, ZR)], zbuf)
            pltpu.sync_copy(zbuf, out_hbm.at[pl.ds(c * N + k * ZR, ZR)])


def _agg_body(x_hbm, src_hbm, dst_hbm, out_hbm, isrc, idst, rows0, rows1,
              sem0, sem1, agg_s):
    c = lax.axis_index("c")
    s = lax.axis_index("s")
    wid = c * NS + s

    zbuf = rows0.at[pl.ds(0, ZR)]
    _fill_const(rows0, ZR, D, 0.0)
    _zero_spmem(s, zbuf, agg_s)

    pltpu.sync_copy(src_hbm.at[pl.ds(wid * EPT, EPT)], isrc)
    pltpu.sync_copy(dst_hbm.at[wid], idst)
    plsc.subcore_barrier()

    cp0 = pltpu.async_copy(x_hbm.at[isrc.at[pl.ds(0, CH)]], rows0, sem0)
    cp1 = pltpu.async_copy(x_hbm.at[isrc.at[pl.ds(CH, CH)]], rows1, sem1)

    def pair(i, _):
        k = 2 * i
        cp0.wait()
        pltpu.sync_copy(rows0, agg_s.at[idst.at[k]], add=True)

        @pl.when(k + 2 < NCH)
        def _():
            pltpu.async_copy(x_hbm.at[isrc.at[pl.ds((k + 2) * CH, CH)]],
                             rows0, sem0)

        cp1.wait()
        pltpu.sync_copy(rows1, agg_s.at[idst.at[k + 1]], add=True)

        @pl.when(k + 3 < NCH)
        def _():
            pltpu.async_copy(x_hbm.at[isrc.at[pl.ds((k + 3) * CH, CH)]],
                             rows1, sem1)

        return 0

    lax.fori_loop(0, NCH // 2, pair, 0)
    if NCH % 2 == 1:
        cp0.wait()
        pltpu.sync_copy(rows0, agg_s.at[idst.at[NCH - 1]], add=True)
    plsc.subcore_barrier()
    _export_spmem(c, s, rows0.at[pl.ds(0, ZR)], agg_s, out_hbm)


_SC_MESH = plsc.VectorSubcoreMesh(core_axis_name="c", subcore_axis_name="s")

_agg = pl.kernel(
    _agg_body,
    out_type=(jax.ShapeDtypeStruct((NC * N, D), jnp.float32),),
    mesh=_SC_MESH,
    scratch_types=[
        pltpu.VMEM((EPT,), jnp.int32),
        pltpu.VMEM((NCH, CH), jnp.int32),
        pltpu.VMEM((CH, D), jnp.float32),
        pltpu.VMEM((CH, D), jnp.float32),
        pltpu.SemaphoreType.DMA,
        pltpu.SemaphoreType.DMA,
        pltpu.VMEM_SHARED((N, D), jnp.float32),
    ],
)

BN = 1000


def _dense1_body(p0, p1, d0, d1, h, wl, b, wr, o, oinv):
    deg = (d0[...] + d1[...])[:, 0:1]
    inv = 1.0 / jnp.maximum(deg, 1.0)
    oinv[...] = inv
    agg = (p0[...] + p1[...]) * inv
    acc = jnp.dot(agg, wl[...], preferred_element_type=jnp.float32)
    acc += jnp.dot(h[...], wr[...], preferred_element_type=jnp.float32)
    o[...] = jnp.maximum(acc + b[...], 0.0)


def _dense_body(relu, p0, p1, dinv, h, wl, b, wr, o):
    agg = (p0[...] + p1[...]) * dinv[...]
    acc = jnp.dot(agg, wl[...], preferred_element_type=jnp.float32)
    acc += jnp.dot(h[...], wr[...], preferred_element_type=jnp.float32)
    acc += b[...]
    o[...] = jnp.maximum(acc, 0.0) if relu else acc


def _part_specs():
    return [
        pl.BlockSpec((BN, D), lambda i: (i, 0)),
        pl.BlockSpec((BN, D), lambda i: (i + N // BN, 0)),
    ]


_W_SPECS = [
    pl.BlockSpec((D, D), lambda i: (0, 0)),
    pl.BlockSpec((1, D), lambda i: (0, 0)),
    pl.BlockSpec((D, D), lambda i: (0, 0)),
]

_dense1 = pl.pallas_call(
    _dense1_body,
    grid=(N // BN,),
    in_specs=_part_specs() + _part_specs() + [
        pl.BlockSpec((BN, D), lambda i: (i, 0)),
    ] + _W_SPECS,
    out_specs=[
        pl.BlockSpec((BN, D), lambda i: (i, 0)),
        pl.BlockSpec((BN, 1), lambda i: (i, 0)),
    ],
    out_shape=[
        jax.ShapeDtypeStruct((N, D), jnp.float32),
        jax.ShapeDtypeStruct((N, 1), jnp.float32),
    ],
)


def _make_dense(relu):
    return pl.pallas_call(
        functools.partial(_dense_body, relu),
        grid=(N // BN,),
        in_specs=_part_specs() + [
            pl.BlockSpec((BN, 1), lambda i: (i, 0)),
            pl.BlockSpec((BN, D), lambda i: (i, 0)),
        ] + _W_SPECS,
        out_specs=pl.BlockSpec((BN, D), lambda i: (i, 0)),
        out_shape=jax.ShapeDtypeStruct((N, D), jnp.float32),
    )


_dense_relu = _make_dense(True)
_dense_lin = _make_dense(False)


def kernel(x, edge_index, c1_Wl, c1_b, c1_Wr, c2_Wl, c2_b, c2_Wr,
           c3_Wl, c3_b, c3_Wr, c4a_Wl, c4a_b, c4a_Wr, c4s_Wl, c4s_b,
           c4s_Wr, c4e_Wl, c4e_b, c4e_Wr):
    src = edge_index[0]
    dst = edge_index[1].reshape(NW, NCH, CH)

    degp = _agg(jnp.ones((N, D), jnp.float32), src, dst)[0]
    parts = _agg(x, src, dst)[0]
    h1, inv = _dense1(parts, parts, degp, degp, x,
                      c1_Wl, c1_b.reshape(1, D), c1_Wr)

    parts = _agg(h1, src, dst)[0]
    h2 = _dense_relu(parts, parts, inv, h1, c2_Wl, c2_b.reshape(1, D), c2_Wr)

    parts = _agg(h2, src, dst)[0]
    h3 = _dense_relu(parts, parts, inv, h2, c3_Wl, c3_b.reshape(1, D), c3_Wr)

    F = 21 + 2 + 5
    wl4 = jnp.zeros((D, D), jnp.float32)
    wl4 = wl4.at[:, :F].set(jnp.concatenate([c4a_Wl, c4s_Wl, c4e_Wl], axis=1))
    wr4 = jnp.zeros((D, D), jnp.float32)
    wr4 = wr4.at[:, :F].set(jnp.concatenate([c4a_Wr, c4s_Wr, c4e_Wr], axis=1))
    b4 = jnp.zeros((1, D), jnp.float32)
    b4 = b4.at[0, :F].set(jnp.concatenate([c4a_b, c4s_b, c4e_b]))

    parts = _agg(h3, src, dst)[0]
    out4 = _dense_lin(parts, parts, inv, h3, wl4, b4, wr4)
    return (out4[:, :21], out4[:, 21:23], out4[:, 23:28])


# speedup vs baseline: 1.3019x; 1.0373x over previous
import functools
import jax
import jax.numpy as jnp
from jax import lax
from jax.experimental import pallas as pl
from jax.experimental.pallas import tpu as pltpu
from jax.experimental.pallas import tpu_sc as plsc

N = 10000
E = 320000
D = 128
NC = 2
NS = 16
NW = NC * NS
CH = 80
EPT = E // NW
NCH = EPT // CH
ZR = 80
NZCH = N // ZR


def _fill_const(buf, rows, cols, val):
    v = jnp.full((16,), val, jnp.float32)

    def row(r, _):
        for k in range(cols // 16):
            buf[r, pl.ds(k * 16, 16)] = v
        return 0

    lax.fori_loop(0, rows, row, 0)


_NJ = (NZCH + NS - 1) // NS


def _zero_spmem(s, zbuf, agg_s, sem):
    cp = None
    for j in range(_NJ - 1):
        cp = pltpu.async_copy(zbuf, agg_s.at[pl.ds((s + j * NS) * ZR, ZR)],
                              sem)

    @pl.when(s + (_NJ - 1) * NS < NZCH)
    def _():
        pltpu.async_copy(
            zbuf, agg_s.at[pl.ds((s + (_NJ - 1) * NS) * ZR, ZR)],
            sem).wait()

    for j in range(_NJ - 1):
        cp.wait()


def _export_spmem(c, s, bufs, sems, agg_s, out_hbm):
    for j in range(_NJ):
        k = s + j * NS
        b = j % 2

        def body(k=k, b=b, j=j):
            if j >= 2:
                pltpu.make_async_copy(
                    bufs[b], out_hbm.at[pl.ds(0, ZR)], sems[b]).wait()
            pltpu.sync_copy(agg_s.at[pl.ds(k * ZR, ZR)], bufs[b])
            pltpu.async_copy(bufs[b],
                             out_hbm.at[pl.ds(c * N + k * ZR, ZR)], sems[b])

        if j == _NJ - 1:
            pl.when(k < NZCH)(body)
        else:
            body()

    pltpu.make_async_copy(bufs[0], out_hbm.at[pl.ds(0, ZR)], sems[0]).wait()

    @pl.when(s + (_NJ - 1) * NS < NZCH)
    def _():
        pltpu.make_async_copy(bufs[1], out_hbm.at[pl.ds(0, ZR)],
                              sems[1]).wait()


def _agg_body(x_hbm, src_hbm, dst_hbm, out_hbm, isrc, idst, rows0, rows1,
              sem0, sem1, agg_s):
    c = lax.axis_index("c")
    s = lax.axis_index("s")
    wid = c * NS + s

    cpi0 = pltpu.async_copy(src_hbm.at[pl.ds(wid * EPT, EPT)], isrc, sem1)
    cpi1 = pltpu.async_copy(dst_hbm.at[wid], idst, sem1)

    zbuf = rows0.at[pl.ds(0, ZR)]
    _fill_const(rows0, ZR, D, 0.0)
    _zero_spmem(s, zbuf, agg_s, sem0)
    cpi0.wait()
    cpi1.wait()
    plsc.subcore_barrier()

    cp0 = pltpu.async_copy(x_hbm.at[isrc.at[pl.ds(0, CH)]], rows0, sem0)
    cp1 = pltpu.async_copy(x_hbm.at[isrc.at[pl.ds(CH, CH)]], rows1, sem1)

    def pair(i, _):
        k = 2 * i
        cp0.wait()
        pltpu.sync_copy(rows0, agg_s.at[idst.at[k]], add=True)

        @pl.when(k + 2 < NCH)
        def _():
            pltpu.async_copy(x_hbm.at[isrc.at[pl.ds((k + 2) * CH, CH)]],
                             rows0, sem0)

        cp1.wait()
        pltpu.sync_copy(rows1, agg_s.at[idst.at[k + 1]], add=True)

        @pl.when(k + 3 < NCH)
        def _():
            pltpu.async_copy(x_hbm.at[isrc.at[pl.ds((k + 3) * CH, CH)]],
                             rows1, sem1)

        return 0

    lax.fori_loop(0, NCH // 2, pair, 0)
    if NCH % 2 == 1:
        cp0.wait()
        pltpu.sync_copy(rows0, agg_s.at[idst.at[NCH - 1]], add=True)
    plsc.subcore_barrier()
    _export_spmem(c, s, (rows0, rows1), (sem0, sem1), agg_s, out_hbm)


_SC_MESH = plsc.VectorSubcoreMesh(core_axis_name="c", subcore_axis_name="s")

_agg = pl.kernel(
    _agg_body,
    out_type=(jax.ShapeDtypeStruct((NC * N, D), jnp.float32),),
    mesh=_SC_MESH,
    scratch_types=[
        pltpu.VMEM((EPT,), jnp.int32),
        pltpu.VMEM((NCH, CH), jnp.int32),
        pltpu.VMEM((CH, D), jnp.float32),
        pltpu.VMEM((CH, D), jnp.float32),
        pltpu.SemaphoreType.DMA,
        pltpu.SemaphoreType.DMA,
        pltpu.VMEM_SHARED((N, D), jnp.float32),
    ],
)

BN = 1000


def _dense1_body(p0, p1, d0, d1, h, wl, b, wr, o, oinv):
    deg = (d0[...] + d1[...])[:, 0:1]
    inv = 1.0 / jnp.maximum(deg, 1.0)
    oinv[...] = inv
    agg = (p0[...] + p1[...]) * inv
    acc = jnp.dot(agg, wl[...], preferred_element_type=jnp.float32)
    acc += jnp.dot(h[...], wr[...], preferred_element_type=jnp.float32)
    o[...] = jnp.maximum(acc + b[...], 0.0)


def _dense_body(relu, p0, p1, dinv, h, wl, b, wr, o):
    agg = (p0[...] + p1[...]) * dinv[...]
    acc = jnp.dot(agg, wl[...], preferred_element_type=jnp.float32)
    acc += jnp.dot(h[...], wr[...], preferred_element_type=jnp.float32)
    acc += b[...]
    o[...] = jnp.maximum(acc, 0.0) if relu else acc


def _part_specs():
    return [
        pl.BlockSpec((BN, D), lambda i: (i, 0)),
        pl.BlockSpec((BN, D), lambda i: (i + N // BN, 0)),
    ]


_W_SPECS = [
    pl.BlockSpec((D, D), lambda i: (0, 0)),
    pl.BlockSpec((1, D), lambda i: (0, 0)),
    pl.BlockSpec((D, D), lambda i: (0, 0)),
]

_dense1 = pl.pallas_call(
    _dense1_body,
    grid=(N // BN,),
    in_specs=_part_specs() + _part_specs() + [
        pl.BlockSpec((BN, D), lambda i: (i, 0)),
    ] + _W_SPECS,
    out_specs=[
        pl.BlockSpec((BN, D), lambda i: (i, 0)),
        pl.BlockSpec((BN, 1), lambda i: (i, 0)),
    ],
    out_shape=[
        jax.ShapeDtypeStruct((N, D), jnp.float32),
        jax.ShapeDtypeStruct((N, 1), jnp.float32),
    ],
)


def _make_dense(relu):
    return pl.pallas_call(
        functools.partial(_dense_body, relu),
        grid=(N // BN,),
        in_specs=_part_specs() + [
            pl.BlockSpec((BN, 1), lambda i: (i, 0)),
            pl.BlockSpec((BN, D), lambda i: (i, 0)),
        ] + _W_SPECS,
        out_specs=pl.BlockSpec((BN, D), lambda i: (i, 0)),
        out_shape=jax.ShapeDtypeStruct((N, D), jnp.float32),
    )


_dense_relu = _make_dense(True)
_dense_lin = _make_dense(False)


def kernel(x, edge_index, c1_Wl, c1_b, c1_Wr, c2_Wl, c2_b, c2_Wr,
           c3_Wl, c3_b, c3_Wr, c4a_Wl, c4a_b, c4a_Wr, c4s_Wl, c4s_b,
           c4s_Wr, c4e_Wl, c4e_b, c4e_Wr):
    src = edge_index[0]
    dst = edge_index[1].reshape(NW, NCH, CH)

    degp = _agg(jnp.ones((N, D), jnp.float32), src, dst)[0]
    parts = _agg(x, src, dst)[0]
    h1, inv = _dense1(parts, parts, degp, degp, x,
                      c1_Wl, c1_b.reshape(1, D), c1_Wr)

    parts = _agg(h1, src, dst)[0]
    h2 = _dense_relu(parts, parts, inv, h1, c2_Wl, c2_b.reshape(1, D), c2_Wr)

    parts = _agg(h2, src, dst)[0]
    h3 = _dense_relu(parts, parts, inv, h2, c3_Wl, c3_b.reshape(1, D), c3_Wr)

    F = 21 + 2 + 5
    wl4 = jnp.zeros((D, D), jnp.float32)
    wl4 = wl4.at[:, :F].set(jnp.concatenate([c4a_Wl, c4s_Wl, c4e_Wl], axis=1))
    wr4 = jnp.zeros((D, D), jnp.float32)
    wr4 = wr4.at[:, :F].set(jnp.concatenate([c4a_Wr, c4s_Wr, c4e_Wr], axis=1))
    b4 = jnp.zeros((1, D), jnp.float32)
    b4 = b4.at[0, :F].set(jnp.concatenate([c4a_b, c4s_b, c4e_b]))

    parts = _agg(h3, src, dst)[0]
    out4 = _dense_lin(parts, parts, inv, h3, wl4, b4, wr4)
    return (out4[:, :21], out4[:, 21:23], out4[:, 23:28])


# speedup vs baseline: 1.4012x; 1.0763x over previous
import functools
import jax
import jax.numpy as jnp
from jax import lax
from jax.experimental import pallas as pl
from jax.experimental.pallas import tpu as pltpu
from jax.experimental.pallas import tpu_sc as plsc

N = 10000
E = 320000
D = 128
NC = 2
NS = 16
NW = NC * NS
CH = 80
EPT = E // NW
NCH = EPT // CH
ZR = 80
NZCH = N // ZR


def _fill_const(buf, rows, cols, val):
    v = jnp.full((16,), val, jnp.float32)

    def row(r, _):
        for k in range(cols // 16):
            buf[r, pl.ds(k * 16, 16)] = v
        return 0

    lax.fori_loop(0, rows, row, 0)


_NJ = (NZCH + NS - 1) // NS


def _zero_spmem(s, zbuf, agg_s, sem):
    cp = None
    for j in range(_NJ - 1):
        cp = pltpu.async_copy(zbuf, agg_s.at[pl.ds((s + j * NS) * ZR, ZR)],
                              sem)

    @pl.when(s + (_NJ - 1) * NS < NZCH)
    def _():
        pltpu.async_copy(
            zbuf, agg_s.at[pl.ds((s + (_NJ - 1) * NS) * ZR, ZR)],
            sem).wait()

    for j in range(_NJ - 1):
        cp.wait()


def _export_spmem(c, s, bufs, sems, agg_s, out_hbm):
    for j in range(_NJ):
        k = s + j * NS
        b = j % 2

        def body(k=k, b=b, j=j):
            if j >= 2:
                pltpu.make_async_copy(
                    bufs[b], out_hbm.at[pl.ds(0, ZR)], sems[b]).wait()
            pltpu.sync_copy(agg_s.at[pl.ds(k * ZR, ZR)], bufs[b])
            pltpu.async_copy(bufs[b],
                             out_hbm.at[pl.ds(c * N + k * ZR, ZR)], sems[b])

        if j == _NJ - 1:
            pl.when(k < NZCH)(body)
        else:
            body()

    pltpu.make_async_copy(bufs[0], out_hbm.at[pl.ds(0, ZR)], sems[0]).wait()

    @pl.when(s + (_NJ - 1) * NS < NZCH)
    def _():
        pltpu.make_async_copy(bufs[1], out_hbm.at[pl.ds(0, ZR)],
                              sems[1]).wait()


def _agg_body(x_hbm, src_hbm, dst_hbm, mode_hbm, out_hbm, isrc, idst,
              rows0, rows1, mode_v, sem0, sem1, agg_s):
    c = lax.axis_index("c")
    s = lax.axis_index("s")
    wid = c * NS + s

    cpi0 = pltpu.async_copy(src_hbm.at[pl.ds(wid * EPT, EPT)], isrc, sem1)
    cpi1 = pltpu.async_copy(dst_hbm.at[wid], idst, sem1)
    cpm = pltpu.async_copy(mode_hbm, mode_v, sem1)

    zbuf = rows0.at[pl.ds(0, ZR)]
    _fill_const(rows0, ZR, D, 0.0)
    _zero_spmem(s, zbuf, agg_s, sem0)
    cpi0.wait()
    cpi1.wait()
    cpm.wait()
    do_gather = mode_v[...][0] == 0

    @pl.when(jnp.logical_not(do_gather))
    def _():
        _fill_const(rows0, CH, D, 1.0)
        _fill_const(rows1, CH, D, 1.0)

    plsc.subcore_barrier()

    cp0 = pltpu.make_async_copy(x_hbm.at[isrc.at[pl.ds(0, CH)]], rows0, sem0)
    cp1 = pltpu.make_async_copy(x_hbm.at[isrc.at[pl.ds(CH, CH)]], rows1, sem1)

    @pl.when(do_gather)
    def _():
        cp0.start()
        cp1.start()

    def pair(i, _):
        k = 2 * i

        @pl.when(do_gather)
        def _():
            cp0.wait()

        pltpu.sync_copy(rows0, agg_s.at[idst.at[k]], add=True)

        @pl.when(jnp.logical_and(do_gather, k + 2 < NCH))
        def _():
            pltpu.async_copy(x_hbm.at[isrc.at[pl.ds((k + 2) * CH, CH)]],
                             rows0, sem0)

        @pl.when(do_gather)
        def _():
            cp1.wait()

        pltpu.sync_copy(rows1, agg_s.at[idst.at[k + 1]], add=True)

        @pl.when(jnp.logical_and(do_gather, k + 3 < NCH))
        def _():
            pltpu.async_copy(x_hbm.at[isrc.at[pl.ds((k + 3) * CH, CH)]],
                             rows1, sem1)

        return 0

    lax.fori_loop(0, NCH // 2, pair, 0)
    if NCH % 2 == 1:
        @pl.when(do_gather)
        def _():
            cp0.wait()

        pltpu.sync_copy(rows0, agg_s.at[idst.at[NCH - 1]], add=True)
    plsc.subcore_barrier()
    _export_spmem(c, s, (rows0.at[pl.ds(0, ZR)], rows1.at[pl.ds(0, ZR)]),
                  (sem0, sem1), agg_s, out_hbm)


_SC_MESH = plsc.VectorSubcoreMesh(core_axis_name="c", subcore_axis_name="s")

_agg = pl.kernel(
    _agg_body,
    out_type=(jax.ShapeDtypeStruct((NC * N, D), jnp.float32),),
    mesh=_SC_MESH,
    scratch_types=[
        pltpu.VMEM((EPT,), jnp.int32),
        pltpu.VMEM((NCH, CH), jnp.int32),
        pltpu.VMEM((CH, D), jnp.float32),
        pltpu.VMEM((CH, D), jnp.float32),
        pltpu.VMEM((16,), jnp.int32),
        pltpu.SemaphoreType.DMA,
        pltpu.SemaphoreType.DMA,
        pltpu.VMEM_SHARED((N, D), jnp.float32),
    ],
)

BN = 1000


def _dense1_body(p0, p1, d0, d1, h, wl, b, wr, o, oinv):
    deg = (d0[...] + d1[...])[:, 0:1]
    inv = 1.0 / jnp.maximum(deg, 1.0)
    oinv[...] = inv
    agg = (p0[...] + p1[...]) * inv
    acc = jnp.dot(agg, wl[...], preferred_element_type=jnp.float32)
    acc += jnp.dot(h[...], wr[...], preferred_element_type=jnp.float32)
    o[...] = jnp.maximum(acc + b[...], 0.0)


def _dense_body(relu, p0, p1, dinv, h, wl, b, wr, o):
    agg = (p0[...] + p1[...]) * dinv[...]
    acc = jnp.dot(agg, wl[...], preferred_element_type=jnp.float32)
    acc += jnp.dot(h[...], wr[...], preferred_element_type=jnp.float32)
    acc += b[...]
    o[...] = jnp.maximum(acc, 0.0) if relu else acc


def _part_specs():
    return [
        pl.BlockSpec((BN, D), lambda i: (i, 0)),
        pl.BlockSpec((BN, D), lambda i: (i + N // BN, 0)),
    ]


_W_SPECS = [
    pl.BlockSpec((D, D), lambda i: (0, 0)),
    pl.BlockSpec((1, D), lambda i: (0, 0)),
    pl.BlockSpec((D, D), lambda i: (0, 0)),
]

_dense1 = pl.pallas_call(
    _dense1_body,
    grid=(N // BN,),
    in_specs=_part_specs() + _part_specs() + [
        pl.BlockSpec((BN, D), lambda i: (i, 0)),
    ] + _W_SPECS,
    out_specs=[
        pl.BlockSpec((BN, D), lambda i: (i, 0)),
        pl.BlockSpec((BN, 1), lambda i: (i, 0)),
    ],
    out_shape=[
        jax.ShapeDtypeStruct((N, D), jnp.float32),
        jax.ShapeDtypeStruct((N, 1), jnp.float32),
    ],
)


def _make_dense(relu):
    return pl.pallas_call(
        functools.partial(_dense_body, relu),
        grid=(N // BN,),
        in_specs=_part_specs() + [
            pl.BlockSpec((BN, 1), lambda i: (i, 0)),
            pl.BlockSpec((BN, D), lambda i: (i, 0)),
        ] + _W_SPECS,
        out_specs=pl.BlockSpec((BN, D), lambda i: (i, 0)),
        out_shape=jax.ShapeDtypeStruct((N, D), jnp.float32),
    )


_dense_relu = _make_dense(True)
_dense_lin = _make_dense(False)


def kernel(x, edge_index, c1_Wl, c1_b, c1_Wr, c2_Wl, c2_b, c2_Wr,
           c3_Wl, c3_b, c3_Wr, c4a_Wl, c4a_b, c4a_Wr, c4s_Wl, c4s_b,
           c4s_Wr, c4e_Wl, c4e_b, c4e_Wr):
    src = edge_index[0]
    dst = edge_index[1].reshape(NW, NCH, CH)
    m_agg = jnp.zeros((16,), jnp.int32)
    m_deg = jnp.ones((16,), jnp.int32)

    degp = _agg(x, src, dst, m_deg)[0]
    parts = _agg(x, src, dst, m_agg)[0]
    h1, inv = _dense1(parts, parts, degp, degp, x,
                      c1_Wl, c1_b.reshape(1, D), c1_Wr)

    parts = _agg(h1, src, dst, m_agg)[0]
    h2 = _dense_relu(parts, parts, inv, h1, c2_Wl, c2_b.reshape(1, D), c2_Wr)

    parts = _agg(h2, src, dst, m_agg)[0]
    h3 = _dense_relu(parts, parts, inv, h2, c3_Wl, c3_b.reshape(1, D), c3_Wr)

    F = 21 + 2 + 5
    wl4 = jnp.zeros((D, D), jnp.float32)
    wl4 = wl4.at[:, :F].set(jnp.concatenate([c4a_Wl, c4s_Wl, c4e_Wl], axis=1))
    wr4 = jnp.zeros((D, D), jnp.float32)
    wr4 = wr4.at[:, :F].set(jnp.concatenate([c4a_Wr, c4s_Wr, c4e_Wr], axis=1))
    b4 = jnp.zeros((1, D), jnp.float32)
    b4 = b4.at[0, :F].set(jnp.concatenate([c4a_b, c4s_b, c4e_b]))

    parts = _agg(h3, src, dst, m_agg)[0]
    out4 = _dense_lin(parts, parts, inv, h3, wl4, b4, wr4)
    return (out4[:, :21], out4[:, 21:23], out4[:, 23:28])


# speedup vs baseline: 1.4243x; 1.0165x over previous
import functools
import jax
import jax.numpy as jnp
from jax import lax
from jax.experimental import pallas as pl
from jax.experimental.pallas import tpu as pltpu
from jax.experimental.pallas import tpu_sc as plsc

N = 10000
E = 320000
D = 128
NC = 2
NS = 16
NW = NC * NS
CH = 80
EPT = E // NW
NCH = EPT // CH
ZR = 80
NZCH = N // ZR


def _fill_const(buf, rows, cols, val):
    v = jnp.full((16,), val, jnp.float32)

    def row(r, _):
        for k in range(cols // 16):
            buf[r, pl.ds(k * 16, 16)] = v
        return 0

    lax.fori_loop(0, rows, row, 0)


_NJ = (NZCH + NS - 1) // NS


def _zero_spmem(s, zbuf, agg_s, sem):
    cp = None
    for j in range(_NJ - 1):
        cp = pltpu.async_copy(zbuf, agg_s.at[pl.ds((s + j * NS) * ZR, ZR)],
                              sem)

    @pl.when(s + (_NJ - 1) * NS < NZCH)
    def _():
        pltpu.async_copy(
            zbuf, agg_s.at[pl.ds((s + (_NJ - 1) * NS) * ZR, ZR)],
            sem).wait()

    for j in range(_NJ - 1):
        cp.wait()


def _export_spmem(c, s, bufs, sems, agg_s, out_hbm):
    for j in range(_NJ):
        k = s + j * NS
        b = j % 2

        def body(k=k, b=b, j=j):
            if j >= 2:
                pltpu.make_async_copy(
                    bufs[b], out_hbm.at[pl.ds(0, ZR)], sems[b]).wait()
            pltpu.sync_copy(agg_s.at[pl.ds(k * ZR, ZR)], bufs[b])
            pltpu.async_copy(bufs[b],
                             out_hbm.at[pl.ds(c * N + k * ZR, ZR)], sems[b])

        if j == _NJ - 1:
            pl.when(k < NZCH)(body)
        else:
            body()

    pltpu.make_async_copy(bufs[0], out_hbm.at[pl.ds(0, ZR)], sems[0]).wait()

    @pl.when(s + (_NJ - 1) * NS < NZCH)
    def _():
        pltpu.make_async_copy(bufs[1], out_hbm.at[pl.ds(0, ZR)],
                              sems[1]).wait()


def _agg_body(x_hbm, src_hbm, dst_hbm, mode_hbm, out_hbm, isrc, idst,
              rows0, rows1, mode_v, sem0, sem1, agg_s):
    c = lax.axis_index("c")
    s = lax.axis_index("s")
    wid = c * NS + s

    cpi0 = pltpu.async_copy(src_hbm.at[pl.ds(wid * EPT, EPT)], isrc, sem1)
    cpi1 = pltpu.async_copy(dst_hbm.at[wid], idst, sem1)
    cpm = pltpu.async_copy(mode_hbm, mode_v, sem1)

    zbuf = rows0.at[pl.ds(0, ZR)]
    _fill_const(rows0, ZR, D, 0.0)
    _zero_spmem(s, zbuf, agg_s, sem0)
    cpi0.wait()
    cpi1.wait()
    cpm.wait()
    do_gather = mode_v[...][0] == 0

    @pl.when(jnp.logical_not(do_gather))
    def _():
        _fill_const(rows0, CH, D, 1.0)
        _fill_const(rows1, CH, D, 1.0)

    plsc.subcore_barrier()

    cp0 = pltpu.make_async_copy(x_hbm.at[isrc.at[pl.ds(0, CH)]], rows0, sem0)
    cp1 = pltpu.make_async_copy(x_hbm.at[isrc.at[pl.ds(CH, CH)]], rows1, sem1)

    @pl.when(do_gather)
    def _():
        cp0.start()
        cp1.start()

    def pair(i, _):
        k = 2 * i

        @pl.when(do_gather)
        def _():
            cp0.wait()

        pltpu.sync_copy(rows0, agg_s.at[idst.at[k]], add=True)

        @pl.when(jnp.logical_and(do_gather, k + 2 < NCH))
        def _():
            pltpu.async_copy(x_hbm.at[isrc.at[pl.ds((k + 2) * CH, CH)]],
                             rows0, sem0)

        @pl.when(do_gather)
        def _():
            cp1.wait()

        pltpu.sync_copy(rows1, agg_s.at[idst.at[k + 1]], add=True)

        @pl.when(jnp.logical_and(do_gather, k + 3 < NCH))
        def _():
            pltpu.async_copy(x_hbm.at[isrc.at[pl.ds((k + 3) * CH, CH)]],
                             rows1, sem1)

        return 0

    lax.fori_loop(0, NCH // 2, pair, 0)
    if NCH % 2 == 1:
        @pl.when(do_gather)
        def _():
            cp0.wait()

        pltpu.sync_copy(rows0, agg_s.at[idst.at[NCH - 1]], add=True)
    plsc.subcore_barrier()
    _export_spmem(c, s, (rows0.at[pl.ds(0, ZR)], rows1.at[pl.ds(0, ZR)]),
                  (sem0, sem1), agg_s, out_hbm)


_SC_MESH = plsc.VectorSubcoreMesh(core_axis_name="c", subcore_axis_name="s")

_agg = pl.kernel(
    _agg_body,
    out_type=(jax.ShapeDtypeStruct((NC * N, D), jnp.float32),),
    mesh=_SC_MESH,
    scratch_types=[
        pltpu.VMEM((EPT,), jnp.int32),
        pltpu.VMEM((NCH, CH), jnp.int32),
        pltpu.VMEM((CH, D), jnp.float32),
        pltpu.VMEM((CH, D), jnp.float32),
        pltpu.VMEM((16,), jnp.int32),
        pltpu.SemaphoreType.DMA,
        pltpu.SemaphoreType.DMA,
        pltpu.VMEM_SHARED((N, D), jnp.float32),
    ],
)

BN = 2000


def _dense1_body(p0, p1, d0, d1, h, wl, b, wr, o, oinv):
    deg = (d0[...] + d1[...])[:, 0:1]
    inv = 1.0 / jnp.maximum(deg, 1.0)
    oinv[...] = inv
    agg = (p0[...] + p1[...]) * inv
    acc = jnp.dot(agg, wl[...], preferred_element_type=jnp.float32)
    acc += jnp.dot(h[...], wr[...], preferred_element_type=jnp.float32)
    o[...] = jnp.maximum(acc + b[...], 0.0)


def _dense_body(relu, p0, p1, dinv, h, wl, b, wr, o):
    agg = (p0[...] + p1[...]) * dinv[...]
    acc = jnp.dot(agg, wl[...], preferred_element_type=jnp.float32)
    acc += jnp.dot(h[...], wr[...], preferred_element_type=jnp.float32)
    acc += b[...]
    o[...] = jnp.maximum(acc, 0.0) if relu else acc


def _part_specs():
    return [
        pl.BlockSpec((BN, D), lambda i: (i, 0)),
        pl.BlockSpec((BN, D), lambda i: (i + N // BN, 0)),
    ]


_W_SPECS = [
    pl.BlockSpec((D, D), lambda i: (0, 0)),
    pl.BlockSpec((1, D), lambda i: (0, 0)),
    pl.BlockSpec((D, D), lambda i: (0, 0)),
]

_dense1 = pl.pallas_call(
    _dense1_body,
    grid=(N // BN,),
    in_specs=_part_specs() + _part_specs() + [
        pl.BlockSpec((BN, D), lambda i: (i, 0)),
    ] + _W_SPECS,
    out_specs=[
        pl.BlockSpec((BN, D), lambda i: (i, 0)),
        pl.BlockSpec((BN, 1), lambda i: (i, 0)),
    ],
    out_shape=[
        jax.ShapeDtypeStruct((N, D), jnp.float32),
        jax.ShapeDtypeStruct((N, 1), jnp.float32),
    ],
)


def _make_dense(relu):
    return pl.pallas_call(
        functools.partial(_dense_body, relu),
        grid=(N // BN,),
        in_specs=_part_specs() + [
            pl.BlockSpec((BN, 1), lambda i: (i, 0)),
            pl.BlockSpec((BN, D), lambda i: (i, 0)),
        ] + _W_SPECS,
        out_specs=pl.BlockSpec((BN, D), lambda i: (i, 0)),
        out_shape=jax.ShapeDtypeStruct((N, D), jnp.float32),
    )


_dense_relu = _make_dense(True)
_dense_lin = _make_dense(False)


def kernel(x, edge_index, c1_Wl, c1_b, c1_Wr, c2_Wl, c2_b, c2_Wr,
           c3_Wl, c3_b, c3_Wr, c4a_Wl, c4a_b, c4a_Wr, c4s_Wl, c4s_b,
           c4s_Wr, c4e_Wl, c4e_b, c4e_Wr):
    src = edge_index[0]
    dst = edge_index[1].reshape(NW, NCH, CH)
    m_agg = jnp.zeros((16,), jnp.int32)
    m_deg = jnp.ones((16,), jnp.int32)

    degp = _agg(x, src, dst, m_deg)[0]
    parts = _agg(x, src, dst, m_agg)[0]
    h1, inv = _dense1(parts, parts, degp, degp, x,
                      c1_Wl, c1_b.reshape(1, D), c1_Wr)

    parts = _agg(h1, src, dst, m_agg)[0]
    h2 = _dense_relu(parts, parts, inv, h1, c2_Wl, c2_b.reshape(1, D), c2_Wr)

    parts = _agg(h2, src, dst, m_agg)[0]
    h3 = _dense_relu(parts, parts, inv, h2, c3_Wl, c3_b.reshape(1, D), c3_Wr)

    F = 21 + 2 + 5
    wl4 = jnp.zeros((D, D), jnp.float32)
    wl4 = wl4.at[:, :F].set(jnp.concatenate([c4a_Wl, c4s_Wl, c4e_Wl], axis=1))
    wr4 = jnp.zeros((D, D), jnp.float32)
    wr4 = wr4.at[:, :F].set(jnp.concatenate([c4a_Wr, c4s_Wr, c4e_Wr], axis=1))
    b4 = jnp.zeros((1, D), jnp.float32)
    b4 = b4.at[0, :F].set(jnp.concatenate([c4a_b, c4s_b, c4e_b]))

    parts = _agg(h3, src, dst, m_agg)[0]
    out4 = _dense_lin(parts, parts, inv, h3, wl4, b4, wr4)
    return (out4[:, :21], out4[:, 21:23], out4[:, 23:28])
